# baseline (speedup 1.0000x reference)
import functools

import jax
import jax.numpy as jnp
from jax import lax
from jax.experimental import pallas as pl
from jax.experimental.pallas import tpu as pltpu
from jax.experimental.pallas import tpu_sc as plsc

_NUM_USERS = 25000
_NUM_ITEMS = 25000
_D = 64
_H = 32
_N_LAYERS = 3
_N_EDGES = 800000
_BATCH = 16384
_N_NODES = _NUM_USERS + _NUM_ITEMS

_NC = 2
_NS = 16
_L = 16

_IL = plsc.PackFormat.INTERLEAVED


def _build(n_nodes, n_users, e_tile, batch, n_layers, h):
    e_chunk = 256
    ec_rows = e_chunk // 128
    n_chunks = e_tile // e_chunk
    rows_per_tile = n_nodes // _NS
    pub_chunk = 125
    n_pub = rows_per_tile // pub_chunk
    b_tile = batch // _NS
    nb_chunks = b_tile // 128

    mesh = plsc.VectorSubcoreMesh(core_axis_name="c", subcore_axis_name="s",
                                  num_cores=_NC, num_subcores=_NS)

    def body(src_hbm, dst_hbm, vals_hbm, t0_hbm, users_hbm, items_hbm,
             zeros_hbm,
             layers_hbm, users_out, items_out,
             acc, sv0, sv1, sv2, dv0, dv1, dv2, vv0, vv1, vv2,
             bf0, bf1, fv0, fv1,
             is0, is1, is2, gs0, gs1, ss0, ss1):
        c = lax.axis_index("c")
        s = lax.axis_index("s")
        sv = (sv0, sv1, sv2)
        dv = (dv0, dv1, dv2)
        vv = (vv0, vv1, vv2)
        bf = (bf0, bf1)
        fv = (fv0, fv1)
        isem = (is0, is1, is2)
        gsem = (gs0, gs1)
        ssem = (ss0, ss1)
        meta_base = s * (e_tile // 128)
        coff = c * n_nodes

        def meta_fire(ci, b):
            sl = pl.ds(meta_base + ci * ec_rows, ec_rows)
            pltpu.async_copy(src_hbm.at[sl], sv[b], isem[b])
            pltpu.async_copy(dst_hbm.at[sl], dv[b], isem[b])
            slv = pl.ds(s * e_tile + ci * e_chunk, e_chunk)
            pltpu.async_copy(vals_hbm.at[slv], vv[b], isem[b])

        def meta_wait(ci, b):
            sl = pl.ds(meta_base + ci * ec_rows, ec_rows)
            pltpu.make_async_copy(src_hbm.at[sl], sv[b], isem[b]).wait()
            pltpu.make_async_copy(dst_hbm.at[sl], dv[b], isem[b]).wait()
            slv = pl.ds(s * e_tile + ci * e_chunk, e_chunk)
            pltpu.make_async_copy(vals_hbm.at[slv], vv[b], isem[b]).wait()
            for j in range(ec_rows):
                for g in range(128 // _L):
                    sl2 = pl.ds(g * _L, _L)
                    sv[b][j, sl2] = sv[b][j, sl2] + coff

        def gather_fire(tbl, bm, b2):
            for j in range(ec_rows):
                pltpu.async_copy(tbl.at[sv[bm].at[j]],
                                 bf[b2].at[pl.ds(j * 128, 128)], gsem[b2])

        def gather_wait(tbl, bm, b2):
            for j in range(ec_rows):
                pltpu.make_async_copy(tbl.at[sv[bm].at[j]],
                                      bf[b2].at[pl.ds(j * 128, 128)],
                                      gsem[b2]).wait()

        def scatter_fire(bm, b2):
            for j in range(ec_rows):
                pltpu.async_copy(fv[b2].at[pl.ds(j * 128, 128)],
                                 acc.at[dv[bm].at[j]], ssem[b2], add=True)

        def scatter_wait(bm, b2):
            for j in range(ec_rows):
                pltpu.make_async_copy(fv[b2].at[pl.ds(j * 128, 128)],
                                      acc.at[dv[bm].at[j]], ssem[b2]).wait()

        def scale(bm, b2):
            def _scale_body(i, _):
                vals16 = vv[bm][pl.ds(i * _L, _L)]
                r0 = i * _L
                for u in range(_L):
                    val = jnp.broadcast_to(vals16[u], (_L,))
                    r = r0 + u
                    a, b_ = plsc.unpack(bf[b2][r, pl.ds(0, 2 * _L)],
                                        format=_IL)
                    fv[b2][r, pl.ds(0, _L)] = a * val
                    fv[b2][r, pl.ds(_L, _L)] = b_ * val
                return 0
            lax.fori_loop(0, e_chunk // _L, _scale_body, 0)

        for k in range(n_layers):
            tbl = t0_hbm if k == 0 else layers_hbm.at[k - 1]

            pltpu.sync_copy(zeros_hbm,
                            acc.at[pl.ds(s * rows_per_tile, rows_per_tile)])
            plsc.subcore_barrier()

            meta_fire(0, 0)
            meta_fire(1, 1)
            meta_wait(0, 0)
            gather_fire(tbl, 0, 0)

            def step6(p, _):
                for q in range(6):
                    ci = 6 * p + q
                    bm = q % 3
                    b2 = q % 2

                    @pl.when(ci + 1 < n_chunks)
                    def _():
                        meta_wait(ci + 1, (q + 1) % 3)
                        gather_fire(tbl, (q + 1) % 3, (q + 1) % 2)

                    gather_wait(tbl, bm, b2)
                    scale(bm, b2)

                    if q == 0:
                        @pl.when(ci >= 1)
                        def _():
                            scatter_wait((q + 2) % 3, (q + 1) % 2)
                    else:
                        scatter_wait((q + 2) % 3, (q + 1) % 2)

                    scatter_fire(bm, b2)

                    @pl.when(ci + 2 < n_chunks)
                    def _():
                        meta_fire(ci + 2, (q + 2) % 3)
                return 0
            lax.fori_loop(0, n_chunks // 6, step6, 0)
            scatter_wait((n_chunks - 1) % 3, (n_chunks - 1) % 2)
            plsc.subcore_barrier()

            def pub(p, _):
                r0 = s * rows_per_tile + p * pub_chunk
                pltpu.sync_copy(acc.at[pl.ds(r0, pub_chunk)],
                                fv0.at[pl.ds(0, pub_chunk)])

                def conv(r, _):
                    packed = plsc.pack(fv0[r, pl.ds(0, _L)],
                                       fv0[r, pl.ds(_L, _L)], format=_IL)
                    bf0[r, pl.ds(0, 2 * _L)] = packed
                    return 0
                lax.fori_loop(0, pub_chunk, conv, 0)
                pltpu.sync_copy(
                    bf0.at[pl.ds(0, pub_chunk)],
                    layers_hbm.at[k].at[pl.ds(c * n_nodes + r0, pub_chunk)])
                return 0
            lax.fori_loop(0, n_pub, pub, 0)
            plsc.subcore_barrier()

        quarter = jnp.float32(0.25)
        for boff, idx_hbm, out_hbm in ((0, users_hbm, users_out),
                                       (n_users, items_hbm, items_out)):
            def bchunk(j, _):
                row0 = s * nb_chunks + j
                pltpu.sync_copy(idx_hbm.at[row0], sv0.at[0])
                for g in range(128 // _L):
                    sl2 = pl.ds(g * _L, _L)
                    sv0[0, sl2] = sv0[0, sl2] + (coff + boff)
                idx = sv0.at[0]
                hs = [
                    pltpu.async_copy(t0_hbm.at[idx],
                                     bf0.at[pl.ds(0, 128)], gs0),
                    pltpu.async_copy(layers_hbm.at[0].at[idx],
                                     bf0.at[pl.ds(128, 128)], gs0),
                    pltpu.async_copy(layers_hbm.at[1].at[idx],
                                     bf1.at[pl.ds(0, 128)], gs0),
                    pltpu.async_copy(layers_hbm.at[2].at[idx],
                                     bf1.at[pl.ds(128, 128)], gs0),
                ]
                for hh in hs:
                    hh.wait()

                def comb(r, _):
                    a0, b0_ = plsc.unpack(bf0[r, pl.ds(0, 2 * _L)], format=_IL)
                    a1, b1_ = plsc.unpack(bf0[128 + r, pl.ds(0, 2 * _L)],
                                          format=_IL)
                    a2, b2_ = plsc.unpack(bf1[r, pl.ds(0, 2 * _L)], format=_IL)
                    a3, b3_ = plsc.unpack(bf1[128 + r, pl.ds(0, 2 * _L)],
                                          format=_IL)
                    fv0[r, pl.ds(0, _L)] = (a0 + a1 + a2 + a3) * quarter
                    fv0[r, pl.ds(_L, _L)] = (b0_ + b1_ + b2_ + b3_) * quarter
                    return 0
                lax.fori_loop(0, 128, comb, 0)

                out_base = s * b_tile + j * 128
                pltpu.sync_copy(fv0.at[pl.ds(0, 128)],
                                out_hbm.at[pl.ds(out_base, 128),
                                           pl.ds(c * h, h)])
                return 0
            lax.fori_loop(0, nb_chunks, bchunk, 0)

    out_type = (
        jax.ShapeDtypeStruct((n_layers, _NC * n_nodes, h), jnp.bfloat16),
        jax.ShapeDtypeStruct((batch, _NC * h), jnp.float32),
        jax.ShapeDtypeStruct((batch, _NC * h), jnp.float32),
    )
    scratch = (
        [pltpu.VMEM_SHARED((n_nodes, h), jnp.float32)]
        + [pltpu.VMEM((ec_rows, 128), jnp.int32) for _ in range(3)]
        + [pltpu.VMEM((ec_rows, 128), jnp.int32) for _ in range(3)]
        + [pltpu.VMEM((e_chunk,), jnp.float32) for _ in range(3)]
        + [pltpu.VMEM((e_chunk, h), jnp.bfloat16) for _ in range(2)]
        + [pltpu.VMEM((e_chunk, h), jnp.float32) for _ in range(2)]
        + [pltpu.SemaphoreType.DMA for _ in range(7)]
    )
    return pl.kernel(body, out_type=out_type, mesh=mesh, scratch_types=scratch,
                     compiler_params=pltpu.CompilerParams(
                         use_tc_tiling_on_sc=False,
                         needs_layout_passes=False))


def _prep(users, items, edge_index, edge_vals, user_emb, item_emb, e_tile):
    all_emb = jnp.concatenate([user_emb, item_emb], axis=0)
    h = all_emb.shape[1] // 2
    thalf = jnp.concatenate([all_emb[:, :h], all_emb[:, h:]], axis=0)
    t0 = jnp.stack([thalf[:, :h // 2], thalf[:, h // 2:]],
                   axis=2).reshape(-1, h).astype(jnp.bfloat16)
    pad = e_tile * _NS - edge_index.shape[1]
    srcp = jnp.pad(edge_index[0], (0, pad)).reshape(-1, 128)
    dstp = jnp.pad(edge_index[1], (0, pad)).reshape(-1, 128)
    valsp = jnp.pad(edge_vals, (0, pad))
    users_r = users.reshape(-1, 128)
    items_r = items.reshape(-1, 128)
    zeros = jnp.zeros((_N_NODES // _NS, h), jnp.float32)
    return srcp, dstp, valsp, t0, users_r, items_r, zeros


@jax.jit
def kernel(users, items, edge_index, edge_vals, user_emb, item_emb):
    e_tile = 50688
    srcp, dstp, valsp, t0, users_r, items_r, zeros = _prep(
        users, items, edge_index, edge_vals, user_emb, item_emb, e_tile)
    fn = _build(_N_NODES, _NUM_USERS, e_tile, _BATCH, _N_LAYERS, _H)
    _, users_emb, items_emb = fn(srcp, dstp, valsp, t0, users_r, items_r,
                                 zeros)
    return (users_emb, items_emb)
